# baseline (speedup 1.0000x reference)
import functools

import jax
import jax.numpy as jnp
from jax import lax
from jax.experimental import pallas as pl
from jax.experimental.pallas import tpu as pltpu

N, F, H, C, Q = 10000, 128, 16, 16, 2048

BM1 = 400
BR = 512
NB = Q // BR


def _matmul_small_kernel(a_ref, b_ref, o_ref):
    o_ref[...] = jnp.dot(a_ref[...], b_ref[...],
                         preferred_element_type=jnp.float32)


def _layer1_kernel(adj_ref, xw_ref, h_ref):
    acc = jnp.dot(adj_ref[...], xw_ref[...],
                  preferred_element_type=jnp.float32)
    h_ref[...] = jnp.maximum(acc, 0.0)


def _layer2_kernel(idx_ref, adj_hbm, h_ref, w1_ref, out_ref, buf, sem):
    b = pl.program_id(0)

    def start_batch(batch, slot):
        base = batch * BR
        for j in range(BR):
            pltpu.make_async_copy(
                adj_hbm.at[idx_ref[base + j]],
                buf.at[slot, j],
                sem.at[slot],
            ).start()

    def wait_batch(batch, slot):
        pltpu.make_async_copy(
            adj_hbm.at[pl.ds(0, BR)],
            buf.at[slot],
            sem.at[slot],
        ).wait()

    slot = lax.rem(b, 2)

    @pl.when(b == 0)
    def _():
        start_batch(0, 0)

    @pl.when(b + 1 < NB)
    def _():
        start_batch(b + 1, 1 - slot)

    wait_batch(b, slot)
    rows = buf[slot]
    hr = jnp.dot(rows, h_ref[...], preferred_element_type=jnp.float32)
    out_ref[...] = jnp.dot(hr, w1_ref[...],
                           preferred_element_type=jnp.float32)


@functools.partial(jax.jit, static_argnames=())
def kernel(x, adj, idx, W0, W1):
    xw = pl.pallas_call(
        _matmul_small_kernel,
        out_shape=jax.ShapeDtypeStruct((N, H), jnp.float32),
    )(x, W0)

    nm = N // BM1
    h = pl.pallas_call(
        _layer1_kernel,
        grid=(nm,),
        in_specs=[
            pl.BlockSpec((BM1, N), lambda m: (m, 0)),
            pl.BlockSpec((N, H), lambda m: (0, 0)),
        ],
        out_specs=pl.BlockSpec((BM1, H), lambda m: (m, 0)),
        out_shape=jax.ShapeDtypeStruct((N, H), jnp.float32),
    )(adj, xw)

    grid_spec = pltpu.PrefetchScalarGridSpec(
        num_scalar_prefetch=1,
        grid=(NB,),
        in_specs=[
            pl.BlockSpec(memory_space=pl.ANY),
            pl.BlockSpec((N, H), lambda b, idx_ref: (0, 0)),
            pl.BlockSpec((H, C), lambda b, idx_ref: (0, 0)),
        ],
        out_specs=pl.BlockSpec((BR, C), lambda b, idx_ref: (b, 0)),
        scratch_shapes=[
            pltpu.VMEM((2, BR, N), jnp.float32),
            pltpu.SemaphoreType.DMA((2,)),
        ],
    )
    out = pl.pallas_call(
        _layer2_kernel,
        grid_spec=grid_spec,
        out_shape=jax.ShapeDtypeStruct((Q, C), jnp.float32),
    )(idx, adj, h, W1)
    return out


# speedup vs baseline: 1.0426x; 1.0426x over previous
import functools

import jax
import jax.numpy as jnp
from jax import lax
from jax.experimental import pallas as pl
from jax.experimental.pallas import tpu as pltpu

N, F, H, C, Q = 10000, 128, 16, 16, 2048

BM1 = 400
BR = 256
NB = Q // BR


def _matmul_small_kernel(a_ref, b_ref, o_ref):
    o_ref[...] = jnp.dot(a_ref[...], b_ref[...],
                         preferred_element_type=jnp.float32)


def _layer1_kernel(x_ref, w0_ref, adj_ref, h_ref, xw_ref):
    @pl.when(pl.program_id(0) == 0)
    def _():
        xw_ref[...] = jnp.dot(x_ref[...], w0_ref[...],
                              preferred_element_type=jnp.float32)

    acc = jnp.dot(adj_ref[...], xw_ref[...],
                  preferred_element_type=jnp.float32)
    h_ref[...] = jnp.maximum(acc, 0.0)


def _layer2_kernel(idx_ref, adj_hbm, h_ref, w1_ref, out_ref, buf, sem):
    b = pl.program_id(0)

    def start_batch(batch, slot):
        base = batch * BR
        for j in range(BR):
            pltpu.make_async_copy(
                adj_hbm.at[idx_ref[base + j]],
                buf.at[slot, j],
                sem.at[slot],
            ).start()

    def wait_batch(batch, slot):
        pltpu.make_async_copy(
            adj_hbm.at[pl.ds(0, BR)],
            buf.at[slot],
            sem.at[slot],
        ).wait()

    slot = lax.rem(b, 2)

    @pl.when(b == 0)
    def _():
        start_batch(0, 0)

    @pl.when(b + 1 < NB)
    def _():
        start_batch(b + 1, 1 - slot)

    wait_batch(b, slot)
    rows = buf[slot]
    hr = jnp.dot(rows, h_ref[...], preferred_element_type=jnp.float32)
    out_ref[...] = jnp.dot(hr, w1_ref[...],
                           preferred_element_type=jnp.float32)


@functools.partial(jax.jit, static_argnames=())
def kernel(x, adj, idx, W0, W1):
    nm = N // BM1
    h = pl.pallas_call(
        _layer1_kernel,
        grid=(nm,),
        in_specs=[
            pl.BlockSpec((N, F), lambda m: (0, 0)),
            pl.BlockSpec((F, H), lambda m: (0, 0)),
            pl.BlockSpec((BM1, N), lambda m: (m, 0)),
        ],
        out_specs=pl.BlockSpec((BM1, H), lambda m: (m, 0)),
        out_shape=jax.ShapeDtypeStruct((N, H), jnp.float32),
        scratch_shapes=[pltpu.VMEM((N, H), jnp.float32)],
    )(x, W0, adj)

    grid_spec = pltpu.PrefetchScalarGridSpec(
        num_scalar_prefetch=1,
        grid=(NB,),
        in_specs=[
            pl.BlockSpec(memory_space=pl.ANY),
            pl.BlockSpec((N, H), lambda b, idx_ref: (0, 0)),
            pl.BlockSpec((H, C), lambda b, idx_ref: (0, 0)),
        ],
        out_specs=pl.BlockSpec((BR, C), lambda b, idx_ref: (b, 0)),
        scratch_shapes=[
            pltpu.VMEM((2, BR, N), jnp.float32),
            pltpu.SemaphoreType.DMA((2,)),
        ],
    )
    out = pl.pallas_call(
        _layer2_kernel,
        grid_spec=grid_spec,
        out_shape=jax.ShapeDtypeStruct((Q, C), jnp.float32),
    )(idx, adj, h, W1)
    return out


# speedup vs baseline: 1.0657x; 1.0221x over previous
import functools

import jax
import jax.numpy as jnp
from jax import lax
from jax.experimental import pallas as pl
from jax.experimental.pallas import tpu as pltpu

N, F, H, C, Q = 10000, 128, 16, 16, 2048

BM1 = 200
NM = N // BM1
BR = 256
NB = Q // BR


def _fused_kernel(idx_ref, x_ref, w0_ref, w1_ref, adj_blk, adj_hbm,
                  out_ref, xw_ref, h_ref, buf, sem):
    i = pl.program_id(0)

    def start_batch(batch, slot):
        base = batch * BR
        for j in range(BR):
            pltpu.make_async_copy(
                adj_hbm.at[idx_ref[base + j]],
                buf.at[slot, j],
                sem.at[slot],
            ).start()

    def wait_batch(slot):
        pltpu.make_async_copy(
            adj_hbm.at[pl.ds(0, BR)],
            buf.at[slot],
            sem.at[slot],
        ).wait()

    @pl.when(i == 0)
    def _():
        xw_ref[...] = jnp.dot(x_ref[...], w0_ref[...],
                              preferred_element_type=jnp.float32)

    @pl.when(i < NM)
    def _():
        acc = jnp.dot(adj_blk[...], xw_ref[...],
                      preferred_element_type=jnp.float32)
        h_ref[pl.ds(i * BM1, BM1), :] = jnp.maximum(acc, 0.0)

    @pl.when(i == NM - 1)
    def _():
        start_batch(0, 0)

    @pl.when(i >= NM)
    def _():
        b = i - NM
        slot = lax.rem(b, 2)

        @pl.when(b + 1 < NB)
        def _():
            start_batch(b + 1, 1 - slot)

        wait_batch(slot)
        rows = buf[slot]
        hr = jnp.dot(rows, h_ref[...], preferred_element_type=jnp.float32)
        out_ref[...] = jnp.dot(hr, w1_ref[...],
                               preferred_element_type=jnp.float32)


@functools.partial(jax.jit, static_argnames=())
def kernel(x, adj, idx, W0, W1):
    grid_spec = pltpu.PrefetchScalarGridSpec(
        num_scalar_prefetch=1,
        grid=(NM + NB,),
        in_specs=[
            pl.BlockSpec((N, F), lambda i, idx_ref: (0, 0)),
            pl.BlockSpec((F, H), lambda i, idx_ref: (0, 0)),
            pl.BlockSpec((H, C), lambda i, idx_ref: (0, 0)),
            pl.BlockSpec((BM1, N),
                         lambda i, idx_ref: (jnp.minimum(i, NM - 1), 0)),
            pl.BlockSpec(memory_space=pl.ANY),
        ],
        out_specs=pl.BlockSpec(
            (BR, C), lambda i, idx_ref: (jnp.maximum(i - NM, 0), 0)),
        scratch_shapes=[
            pltpu.VMEM((N, H), jnp.float32),
            pltpu.VMEM((N, H), jnp.float32),
            pltpu.VMEM((2, BR, N), jnp.float32),
            pltpu.SemaphoreType.DMA((2,)),
        ],
    )
    out = pl.pallas_call(
        _fused_kernel,
        grid_spec=grid_spec,
        out_shape=jax.ShapeDtypeStruct((Q, C), jnp.float32),
    )(idx, x, W0, W1, adj, adj)
    return out


# speedup vs baseline: 1.0835x; 1.0167x over previous
import functools

import jax
import jax.numpy as jnp
from jax import lax
from jax.experimental import pallas as pl
from jax.experimental.pallas import tpu as pltpu

N, F, H, C, Q = 10000, 128, 16, 16, 2048

BM1 = 200
NM = N // BM1
BR = 128
NB = Q // BR
NS = 3


def _fused_kernel(idx_ref, x_ref, w0_ref, w1_ref, adj_blk, adj_hbm,
                  out_ref, xw_ref, h_ref, buf, sem):
    i = pl.program_id(0)

    def start_batch(batch, slot):
        base = batch * BR
        for j in range(BR):
            pltpu.make_async_copy(
                adj_hbm.at[idx_ref[base + j]],
                buf.at[slot, j],
                sem.at[slot],
            ).start()

    def wait_batch(slot):
        pltpu.make_async_copy(
            adj_hbm.at[pl.ds(0, BR)],
            buf.at[slot],
            sem.at[slot],
        ).wait()

    @pl.when(i == 0)
    def _():
        xw_ref[...] = jnp.dot(x_ref[...], w0_ref[...],
                              preferred_element_type=jnp.float32)

    @pl.when(i < NM)
    def _():
        acc = jnp.dot(adj_blk[...], xw_ref[...],
                      preferred_element_type=jnp.float32)
        h_ref[pl.ds(i * BM1, BM1), :] = jnp.maximum(acc, 0.0)

    @pl.when(i == NM - 2)
    def _():
        start_batch(0, 0)

    @pl.when(i == NM - 1)
    def _():
        start_batch(1, 1)

    @pl.when(i >= NM)
    def _():
        b = i - NM
        slot = lax.rem(b, NS)

        @pl.when(b + 2 < NB)
        def _():
            start_batch(b + 2, lax.rem(b + 2, NS))

        wait_batch(slot)
        rows = buf[slot]
        hr = jnp.dot(rows, h_ref[...], preferred_element_type=jnp.float32)
        out_ref[...] = jnp.dot(hr, w1_ref[...],
                               preferred_element_type=jnp.float32)


@functools.partial(jax.jit, static_argnames=())
def kernel(x, adj, idx, W0, W1):
    grid_spec = pltpu.PrefetchScalarGridSpec(
        num_scalar_prefetch=1,
        grid=(NM + NB,),
        in_specs=[
            pl.BlockSpec((N, F), lambda i, idx_ref: (0, 0)),
            pl.BlockSpec((F, H), lambda i, idx_ref: (0, 0)),
            pl.BlockSpec((H, C), lambda i, idx_ref: (0, 0)),
            pl.BlockSpec((BM1, N),
                         lambda i, idx_ref: (jnp.minimum(i, NM - 1), 0)),
            pl.BlockSpec(memory_space=pl.ANY),
        ],
        out_specs=pl.BlockSpec(
            (BR, C), lambda i, idx_ref: (jnp.maximum(i - NM, 0), 0)),
        scratch_shapes=[
            pltpu.VMEM((N, H), jnp.float32),
            pltpu.VMEM((N, H), jnp.float32),
            pltpu.VMEM((NS, BR, N), jnp.float32),
            pltpu.SemaphoreType.DMA((NS,)),
        ],
    )
    out = pl.pallas_call(
        _fused_kernel,
        grid_spec=grid_spec,
        out_shape=jax.ShapeDtypeStruct((Q, C), jnp.float32),
    )(idx, x, W0, W1, adj, adj)
    return out
